# baseline (speedup 1.0000x reference)
import functools

import jax
import jax.numpy as jnp
from jax import lax
from jax.experimental import pallas as pl
from jax.experimental.pallas import tpu as pltpu
from jax.experimental.pallas import tpu_sc as plsc

EMBED = 64
CHUNK = 512


@functools.lru_cache(maxsize=None)
def _gather_fn(n_idx, num_workers, chunk, nc):
    n_per_w = n_idx // num_workers
    num_chunks = n_per_w // chunk
    mesh = plsc.VectorSubcoreMesh(core_axis_name="c", subcore_axis_name="s")

    @functools.partial(
        pl.kernel,
        mesh=mesh,
        out_type=jax.ShapeDtypeStruct((n_idx, EMBED), jnp.float32),
        scratch_types=[
            pltpu.VMEM((chunk,), jnp.int32),
            pltpu.VMEM((chunk, EMBED), jnp.float32),
            pltpu.SemaphoreType.DMA,
        ],
        compiler_params=pltpu.CompilerParams(use_tc_tiling_on_sc=False),
    )
    def k(idx_hbm, table_hbm, out_hbm, idx_v, rows_v, sem):
        wid = lax.axis_index("s") * nc + lax.axis_index("c")
        base = wid * n_per_w

        def body(i, carry):
            off = base + i * chunk
            pltpu.sync_copy(idx_hbm.at[pl.ds(off, chunk)], idx_v)
            pltpu.async_copy(table_hbm.at[idx_v], rows_v, sem).wait()
            pltpu.sync_copy(rows_v, out_hbm.at[pl.ds(off, chunk)])
            return carry

        lax.fori_loop(0, num_chunks, body, 0)

    return k


def kernel(input_ids, table):
    b, h = input_ids.shape
    n = b * h
    info = plsc.get_sparse_core_info()
    nw = info.num_cores * info.num_subcores
    idx = input_ids.reshape(n).astype(jnp.int32)
    out = _gather_fn(n, nw, CHUNK, info.num_cores)(idx, table)
    return out.reshape(b, h, EMBED)


# speedup vs baseline: 1.0386x; 1.0386x over previous
import functools

import jax
import jax.numpy as jnp
from jax import lax
from jax.experimental import pallas as pl
from jax.experimental.pallas import tpu as pltpu
from jax.experimental.pallas import tpu_sc as plsc

EMBED = 64
CHUNK = 800
NBUF = 2


@functools.lru_cache(maxsize=None)
def _gather_fn(n_idx, num_workers, chunk, nc):
    n_per_w = n_idx // num_workers
    num_chunks = n_per_w // chunk
    assert num_chunks % NBUF == 0
    mesh = plsc.VectorSubcoreMesh(core_axis_name="c", subcore_axis_name="s")

    scratch = (
        [pltpu.VMEM((chunk,), jnp.int32) for _ in range(NBUF)]
        + [pltpu.VMEM((chunk, EMBED), jnp.float32) for _ in range(NBUF)]
        + [pltpu.SemaphoreType.DMA for _ in range(3 * NBUF)]
    )

    @functools.partial(
        pl.kernel,
        mesh=mesh,
        out_type=jax.ShapeDtypeStruct((n_idx, EMBED), jnp.float32),
        scratch_types=scratch,
        compiler_params=pltpu.CompilerParams(use_tc_tiling_on_sc=False),
    )
    def k(idx_hbm, table_hbm, out_hbm, *scr):
        idx_v = scr[0:NBUF]
        rows_v = scr[NBUF : 2 * NBUF]
        sem_i = scr[2 * NBUF : 2 * NBUF + NBUF]
        sem_g = scr[3 * NBUF : 3 * NBUF + NBUF]
        sem_o = scr[4 * NBUF : 4 * NBUF + NBUF]

        wid = lax.axis_index("s") * nc + lax.axis_index("c")
        base = wid * n_per_w

        def idx_src(i):
            return idx_hbm.at[pl.ds(base + i * chunk, chunk)]

        def out_dst(i):
            return out_hbm.at[pl.ds(base + i * chunk, chunk)]

        for b in range(NBUF):
            pltpu.async_copy(idx_src(b), idx_v[b], sem_i[b])

        for b in range(NBUF):
            pltpu.make_async_copy(idx_src(b), idx_v[b], sem_i[b]).wait()
            pltpu.async_copy(table_hbm.at[idx_v[b]], rows_v[b], sem_g[b])
        for b in range(NBUF):
            pltpu.make_async_copy(table_hbm.at[idx_v[b]], rows_v[b], sem_g[b]).wait()
            pltpu.async_copy(rows_v[b], out_dst(b), sem_o[b])
            if NBUF < num_chunks:
                pltpu.async_copy(idx_src(NBUF + b), idx_v[b], sem_i[b])

        def round_body(g, carry):
            for b in range(NBUF):
                i = g * NBUF + b
                pltpu.make_async_copy(idx_src(0), idx_v[b], sem_i[b]).wait()
                pltpu.make_async_copy(rows_v[b], out_dst(0), sem_o[b]).wait()
                pltpu.async_copy(table_hbm.at[idx_v[b]], rows_v[b], sem_g[b])
            for b in range(NBUF):
                i = g * NBUF + b
                pltpu.make_async_copy(
                    table_hbm.at[idx_v[b]], rows_v[b], sem_g[b]
                ).wait()
                pltpu.async_copy(rows_v[b], out_dst(i), sem_o[b])

                @pl.when(i + NBUF < num_chunks)
                def _():
                    pltpu.async_copy(idx_src(i + NBUF), idx_v[b], sem_i[b])

            return carry

        lax.fori_loop(1, num_chunks // NBUF, round_body, 0)

        for b in range(NBUF):
            pltpu.make_async_copy(rows_v[b], out_dst(0), sem_o[b]).wait()

    return k


def kernel(input_ids, table):
    b, h = input_ids.shape
    n = b * h
    info = plsc.get_sparse_core_info()
    nw = info.num_cores * info.num_subcores
    idx = input_ids.reshape(n).astype(jnp.int32)
    out = _gather_fn(n, nw, CHUNK, info.num_cores)(idx, table)
    return out.reshape(b, h, EMBED)
